# baseline (speedup 1.0000x reference)
import dataclasses
import functools

import jax
import jax.numpy as jnp
from jax import lax
from jax.experimental import pallas as pl
from jax.experimental.pallas import tpu as pltpu
from jax.experimental.pallas import tpu_sc as plsc

N_ELEMS = 16 * 512 * 512
NB = 8191
HI = 16.0
H = NB + 1
HSIZE = 2 * H
NC, NS, LANES = 2, 16, 16
NW = NC * NS
CHUNK = 16384
UNROLL = 16
SCALE = 511.5


def _sc_hist_kernel(x_hbm, t_hbm, cnt_out, sm_out, mx_out, cnt_v, sm_v, mx_v):
    wid = lax.axis_index("s") * NC + lax.axis_index("c")

    ones = jnp.full((LANES,), 1.0, jnp.float32)
    zeros = jnp.zeros((LANES,), jnp.float32)

    @pl.loop(0, HSIZE, step=LANES)
    def _(i):
        cnt_v[pl.ds(i, LANES)] = zeros
        sm_v[pl.ds(i, LANES)] = zeros

    mx_v[...] = zeros

    def body(x_vmem, t_vmem):
        @plsc.parallel_loop(
            0, CHUNK, step=LANES, unroll=UNROLL,
            carry=jnp.zeros((LANES,), jnp.float32),
        )
        def loop(c, mx):
            row = c >> 9
            col = c & 511
            xv = x_vmem[row, pl.ds(col, LANES)]
            tf = t_vmem[row, pl.ds(col, LANES)].astype(jnp.float32)
            e = 1.0 - xv * (tf + tf - 1.0)
            r = jnp.maximum(e, 0.0)
            m = jnp.minimum(r, HI)
            kf = m * SCALE + (tf * float(H) + 1.0)
            k = kf.astype(jnp.int32)
            plsc.addupdate_scatter(cnt_v, [k], ones)
            plsc.addupdate_scatter(sm_v, [k], r)
            return jnp.maximum(mx, r)

        mx_v[...] = jnp.maximum(mx_v[...], loop)

    pltpu.emit_pipeline(
        body,
        grid=(N_ELEMS // CHUNK,),
        in_specs=[
            pl.BlockSpec((CHUNK // 512, 512), lambda i: (i, 0)),
            pl.BlockSpec((CHUNK // 512, 512), lambda i: (i, 0)),
        ],
        core_axis_name=("c", "s"),
        dimension_semantics=(pltpu.PARALLEL,),
    )(x_hbm, t_hbm)

    pltpu.sync_copy(cnt_v, cnt_out.at[wid])
    pltpu.sync_copy(sm_v, sm_out.at[wid])
    pltpu.sync_copy(mx_v, mx_out.at[wid])


def _shift_down(x, k, axis):
    pads = [(0, 0), (0, 0)]
    pads[axis] = (k, 0)
    padded = jnp.pad(x, pads)
    if axis == 0:
        return padded[: x.shape[0], :]
    return padded[:, : x.shape[1]]


def _prefix2d(x):
    for k in (1, 2, 4, 8, 16, 32, 64):
        x = x + _shift_down(x, k, 1)
    rt = x[:, 127:128]
    rti = rt
    k = 1
    while k < x.shape[0]:
        rti = rti + _shift_down(rti, k, 0)
        k *= 2
    return x + (rti - rt)


def _tc_finish_kernel(cnt_ref, sm_ref, mx_ref, out_ref):
    cnt = jnp.sum(cnt_ref[...], axis=0)
    sm = jnp.sum(sm_ref[...], axis=0)
    cn = cnt[:64, :]
    cp = cnt[64:, :]
    sn = sm[:64, :]
    sp = sm[64:, :]
    pref_cn = _prefix2d(cn)
    pref_cp = _prefix2d(cp)
    p_tot = jnp.sum(cp)
    n_tot = jnp.sum(cn)
    ngt = n_tot - pref_cn
    pge = p_tot - pref_cp + cp
    a = p_tot + ngt
    contrib = sp / a + sn * (p_tot - pge) / (a * (a + cn))
    loss = jnp.sum(contrib)
    maxr = jnp.max(mx_ref[...])
    out = jnp.where(p_tot > 0.0, loss, maxr)
    out_ref[...] = out * jnp.ones((1, 1), jnp.float32)


_SC_PARAMS = pltpu.CompilerParams()
if "needs_layout_passes" in pltpu.CompilerParams.__dataclass_fields__:
    _SC_PARAMS = dataclasses.replace(_SC_PARAMS, needs_layout_passes=False)
_SC_PARAMS = dataclasses.replace(_SC_PARAMS, use_tc_tiling_on_sc=True)


@functools.partial(
    pl.kernel,
    compiler_params=_SC_PARAMS,
    out_type=[
        jax.ShapeDtypeStruct((NW, HSIZE), jnp.float32),
        jax.ShapeDtypeStruct((NW, HSIZE), jnp.float32),
        jax.ShapeDtypeStruct((NW, LANES), jnp.float32),
    ],
    mesh=plsc.VectorSubcoreMesh(core_axis_name="c", subcore_axis_name="s"),
    scratch_types=[
        pltpu.VMEM((HSIZE,), jnp.float32),
        pltpu.VMEM((HSIZE,), jnp.float32),
        pltpu.VMEM((LANES,), jnp.float32),
    ],
)
def _sc_hist(x_hbm, t_hbm, cnt_out, sm_out, mx_out, cnt_v, sm_v, mx_v):
    _sc_hist_kernel(x_hbm, t_hbm, cnt_out, sm_out, mx_out, cnt_v, sm_v, mx_v)


_tc_finish = pl.pallas_call(
    _tc_finish_kernel,
    out_shape=jax.ShapeDtypeStruct((1, 1), jnp.float32),
)


def kernel(inputs, targets):
    x = inputs.reshape(-1, 512)
    t = targets.reshape(-1, 512)
    cnt, sm, mx = _sc_hist(x, t)
    loss = _tc_finish(
        cnt.reshape(NW, 128, 128), sm.reshape(NW, 128, 128), mx
    )
    return loss[0, 0]


# speedup vs baseline: 1.1120x; 1.1120x over previous
import dataclasses
import functools

import jax
import jax.numpy as jnp
from jax import lax
from jax.experimental import pallas as pl
from jax.experimental.pallas import tpu as pltpu
from jax.experimental.pallas import tpu_sc as plsc

N_ELEMS = 16 * 512 * 512
NB = 8191
HI = 16.0
H = NB + 1
HSIZE = 2 * H
NC, NS, LANES = 2, 16, 16
NW = NC * NS
CHUNK = 4096
UNROLL = 8
SCALE = 511.5


def _sc_hist_kernel(x_hbm, t_hbm, cnt_out, sm_out, mx_out, cnt_v, sm_v, mx_v):
    wid = lax.axis_index("s") * NC + lax.axis_index("c")

    ones = jnp.full((LANES,), 1.0, jnp.float32)
    zeros = jnp.zeros((LANES,), jnp.float32)

    @pl.loop(0, HSIZE, step=LANES)
    def _(i):
        cnt_v[pl.ds(i, LANES)] = zeros
        sm_v[pl.ds(i, LANES)] = zeros

    mx_v[...] = zeros

    def body(x_vmem, t_vmem):
        @plsc.parallel_loop(
            0, CHUNK, step=LANES, unroll=UNROLL,
            carry=jnp.zeros((LANES,), jnp.float32),
        )
        def loop(c, mx):
            row = c >> 9
            col = c & 511
            xv = x_vmem[row, pl.ds(col, LANES)]
            tf = t_vmem[row, pl.ds(col, LANES)].astype(jnp.float32)
            e = 1.0 - xv * (tf + tf - 1.0)
            r = jnp.maximum(e, 0.0)
            m = jnp.minimum(r, HI)
            kf = m * SCALE + (tf * float(H) + 1.0)
            k = kf.astype(jnp.int32)
            plsc.addupdate_scatter(cnt_v, [k], ones)
            plsc.addupdate_scatter(sm_v, [k], r)
            return jnp.maximum(mx, r)

        mx_v[...] = jnp.maximum(mx_v[...], loop)

    pltpu.emit_pipeline(
        body,
        grid=(N_ELEMS // CHUNK,),
        in_specs=[
            pl.BlockSpec((CHUNK // 512, 512), lambda i: (i, 0)),
            pl.BlockSpec((CHUNK // 512, 512), lambda i: (i, 0)),
        ],
        core_axis_name=("c", "s"),
        dimension_semantics=(pltpu.PARALLEL,),
    )(x_hbm, t_hbm)

    pltpu.sync_copy(cnt_v, cnt_out.at[wid])
    pltpu.sync_copy(sm_v, sm_out.at[wid])
    pltpu.sync_copy(mx_v, mx_out.at[wid])


def _shift_down(x, k, axis):
    pads = [(0, 0), (0, 0)]
    pads[axis] = (k, 0)
    padded = jnp.pad(x, pads)
    if axis == 0:
        return padded[: x.shape[0], :]
    return padded[:, : x.shape[1]]


def _prefix2d(x):
    for k in (1, 2, 4, 8, 16, 32, 64):
        x = x + _shift_down(x, k, 1)
    rt = x[:, 127:128]
    rti = rt
    k = 1
    while k < x.shape[0]:
        rti = rti + _shift_down(rti, k, 0)
        k *= 2
    return x + (rti - rt)


def _tc_finish_kernel(cnt_ref, sm_ref, mx_ref, out_ref):
    cnt = jnp.sum(cnt_ref[...], axis=0)
    sm = jnp.sum(sm_ref[...], axis=0)
    cn = cnt[:64, :]
    cp = cnt[64:, :]
    sn = sm[:64, :]
    sp = sm[64:, :]
    pref_cn = _prefix2d(cn)
    pref_cp = _prefix2d(cp)
    p_tot = jnp.sum(cp)
    n_tot = jnp.sum(cn)
    ngt = n_tot - pref_cn
    pge = p_tot - pref_cp + cp
    a = p_tot + ngt
    contrib = sp / a + sn * (p_tot - pge) / (a * (a + cn))
    loss = jnp.sum(contrib)
    maxr = jnp.max(mx_ref[...])
    out = jnp.where(p_tot > 0.0, loss, maxr)
    out_ref[...] = out * jnp.ones((1, 1), jnp.float32)


_SC_PARAMS = pltpu.CompilerParams()
if "needs_layout_passes" in pltpu.CompilerParams.__dataclass_fields__:
    _SC_PARAMS = dataclasses.replace(_SC_PARAMS, needs_layout_passes=False)
_SC_PARAMS = dataclasses.replace(_SC_PARAMS, use_tc_tiling_on_sc=True)


@functools.partial(
    pl.kernel,
    compiler_params=_SC_PARAMS,
    out_type=[
        jax.ShapeDtypeStruct((NW, HSIZE), jnp.float32),
        jax.ShapeDtypeStruct((NW, HSIZE), jnp.float32),
        jax.ShapeDtypeStruct((NW, LANES), jnp.float32),
    ],
    mesh=plsc.VectorSubcoreMesh(core_axis_name="c", subcore_axis_name="s"),
    scratch_types=[
        pltpu.VMEM((HSIZE,), jnp.float32),
        pltpu.VMEM((HSIZE,), jnp.float32),
        pltpu.VMEM((LANES,), jnp.float32),
    ],
)
def _sc_hist(x_hbm, t_hbm, cnt_out, sm_out, mx_out, cnt_v, sm_v, mx_v):
    _sc_hist_kernel(x_hbm, t_hbm, cnt_out, sm_out, mx_out, cnt_v, sm_v, mx_v)


_tc_finish = pl.pallas_call(
    _tc_finish_kernel,
    out_shape=jax.ShapeDtypeStruct((1, 1), jnp.float32),
)


def kernel(inputs, targets):
    x = inputs.reshape(-1, 512)
    t = targets.reshape(-1, 512)
    cnt, sm, mx = _sc_hist(x, t)
    loss = _tc_finish(
        cnt.reshape(NW, 128, 128), sm.reshape(NW, 128, 128), mx
    )
    return loss[0, 0]


# speedup vs baseline: 1.1601x; 1.0433x over previous
import dataclasses
import functools

import jax
import jax.numpy as jnp
from jax import lax
from jax.experimental import pallas as pl
from jax.experimental.pallas import tpu as pltpu
from jax.experimental.pallas import tpu_sc as plsc

N_ELEMS = 16 * 512 * 512
NB = 8191
HI = 16.0
H = NB + 1
HSIZE = 2 * H
NC, NS, LANES = 2, 16, 16
NW = NC * NS
CHUNK = 8192
UNROLL = 8
SCALE = 511.5


def _sc_hist_kernel(x_hbm, t_hbm, cnt_out, sm_out, mx_out, cnt_v, sm_v, mx_v):
    wid = lax.axis_index("s") * NC + lax.axis_index("c")

    ones = jnp.full((LANES,), 1.0, jnp.float32)
    zeros = jnp.zeros((LANES,), jnp.float32)

    @pl.loop(0, HSIZE, step=LANES)
    def _(i):
        cnt_v[pl.ds(i, LANES)] = zeros
        sm_v[pl.ds(i, LANES)] = zeros

    mx_v[...] = zeros

    def body(x_vmem, t_vmem):
        @plsc.parallel_loop(
            0, CHUNK, step=LANES, unroll=UNROLL,
            carry=jnp.zeros((LANES,), jnp.float32),
        )
        def loop(c, mx):
            row = c >> 9
            col = c & 511
            xv = x_vmem[row, pl.ds(col, LANES)]
            tf = t_vmem[row, pl.ds(col, LANES)].astype(jnp.float32)
            e = 1.0 - xv * (tf + tf - 1.0)
            r = jnp.maximum(e, 0.0)
            m = jnp.minimum(r, HI)
            kf = m * SCALE + (tf * float(H) + 1.0)
            k = kf.astype(jnp.int32)
            plsc.addupdate_scatter(cnt_v, [k], ones)
            plsc.addupdate_scatter(sm_v, [k], r)
            return jnp.maximum(mx, r)

        mx_v[...] = jnp.maximum(mx_v[...], loop)

    pltpu.emit_pipeline(
        body,
        grid=(N_ELEMS // CHUNK,),
        in_specs=[
            pl.BlockSpec((CHUNK // 512, 512), lambda i: (i, 0)),
            pl.BlockSpec((CHUNK // 512, 512), lambda i: (i, 0)),
        ],
        core_axis_name=("c", "s"),
        dimension_semantics=(pltpu.PARALLEL,),
    )(x_hbm, t_hbm)

    pltpu.sync_copy(cnt_v, cnt_out.at[wid])
    pltpu.sync_copy(sm_v, sm_out.at[wid])
    pltpu.sync_copy(mx_v, mx_out.at[wid])


def _shift_down(x, k, axis):
    pads = [(0, 0), (0, 0)]
    pads[axis] = (k, 0)
    padded = jnp.pad(x, pads)
    if axis == 0:
        return padded[: x.shape[0], :]
    return padded[:, : x.shape[1]]


def _prefix2d(x):
    for k in (1, 2, 4, 8, 16, 32, 64):
        x = x + _shift_down(x, k, 1)
    rt = x[:, 127:128]
    rti = rt
    k = 1
    while k < x.shape[0]:
        rti = rti + _shift_down(rti, k, 0)
        k *= 2
    return x + (rti - rt)


def _tc_finish_kernel(cnt_ref, sm_ref, mx_ref, out_ref):
    cnt = jnp.sum(cnt_ref[...], axis=0)
    sm = jnp.sum(sm_ref[...], axis=0)
    cn = cnt[:64, :]
    cp = cnt[64:, :]
    sn = sm[:64, :]
    sp = sm[64:, :]
    pref_cn = _prefix2d(cn)
    pref_cp = _prefix2d(cp)
    p_tot = jnp.sum(cp)
    n_tot = jnp.sum(cn)
    ngt = n_tot - pref_cn
    pge = p_tot - pref_cp + cp
    a = p_tot + ngt
    contrib = sp / a + sn * (p_tot - pge) / (a * (a + cn))
    loss = jnp.sum(contrib)
    maxr = jnp.max(mx_ref[...])
    out = jnp.where(p_tot > 0.0, loss, maxr)
    out_ref[...] = out * jnp.ones((1, 1), jnp.float32)


_SC_PARAMS = pltpu.CompilerParams()
if "needs_layout_passes" in pltpu.CompilerParams.__dataclass_fields__:
    _SC_PARAMS = dataclasses.replace(_SC_PARAMS, needs_layout_passes=False)
_SC_PARAMS = dataclasses.replace(_SC_PARAMS, use_tc_tiling_on_sc=True)


@functools.partial(
    pl.kernel,
    compiler_params=_SC_PARAMS,
    out_type=[
        jax.ShapeDtypeStruct((NW, HSIZE), jnp.float32),
        jax.ShapeDtypeStruct((NW, HSIZE), jnp.float32),
        jax.ShapeDtypeStruct((NW, LANES), jnp.float32),
    ],
    mesh=plsc.VectorSubcoreMesh(core_axis_name="c", subcore_axis_name="s"),
    scratch_types=[
        pltpu.VMEM((HSIZE,), jnp.float32),
        pltpu.VMEM((HSIZE,), jnp.float32),
        pltpu.VMEM((LANES,), jnp.float32),
    ],
)
def _sc_hist(x_hbm, t_hbm, cnt_out, sm_out, mx_out, cnt_v, sm_v, mx_v):
    _sc_hist_kernel(x_hbm, t_hbm, cnt_out, sm_out, mx_out, cnt_v, sm_v, mx_v)


_tc_finish = pl.pallas_call(
    _tc_finish_kernel,
    out_shape=jax.ShapeDtypeStruct((1, 1), jnp.float32),
)


def kernel(inputs, targets):
    x = inputs.reshape(-1, 512)
    t = targets.reshape(-1, 512)
    cnt, sm, mx = _sc_hist(x, t)
    loss = _tc_finish(
        cnt.reshape(NW, 128, 128), sm.reshape(NW, 128, 128), mx
    )
    return loss[0, 0]


# speedup vs baseline: 1.2088x; 1.0420x over previous
import dataclasses
import functools

import jax
import jax.numpy as jnp
from jax import lax
from jax.experimental import pallas as pl
from jax.experimental.pallas import tpu as pltpu
from jax.experimental.pallas import tpu_sc as plsc

N_ELEMS = 16 * 512 * 512
NB = 8191
HI = 16.0
H = NB + 1
HSIZE = 2 * H
NC, NS, LANES = 2, 16, 16
NW = NC * NS
CHUNK = 8192
UNROLL = 8
SCALE = 511.5


def _sc_hist_kernel(x_hbm, t_hbm, cnt_out, sm_out, mx_out, cnt_v, sm_v, mx_v):
    wid = lax.axis_index("s") * NC + lax.axis_index("c")

    ones = jnp.full((LANES,), 1.0, jnp.float32)
    zeros = jnp.zeros((LANES,), jnp.float32)

    @plsc.parallel_loop(0, HSIZE, step=LANES, unroll=8)
    def _(i):
        cnt_v[pl.ds(i, LANES)] = zeros
        sm_v[pl.ds(i, LANES)] = zeros

    mx_v[...] = zeros

    def body(x_vmem, t_vmem):
        @plsc.parallel_loop(
            0, CHUNK, step=LANES, unroll=UNROLL,
            carry=jnp.zeros((LANES,), jnp.float32),
        )
        def loop(c, mx):
            row = c >> 9
            col = c & 511
            xv = x_vmem[row, pl.ds(col, LANES)]
            tf = t_vmem[row, pl.ds(col, LANES)].astype(jnp.float32)
            e = 1.0 - xv * (tf + tf - 1.0)
            r = jnp.maximum(e, 0.0)
            m = jnp.minimum(r, HI)
            kf = m * SCALE + (tf * float(H) + 1.0)
            k = kf.astype(jnp.int32)
            plsc.addupdate_scatter(cnt_v, [k], ones)
            plsc.addupdate_scatter(sm_v, [k], r)
            return jnp.maximum(mx, r)

        mx_v[...] = jnp.maximum(mx_v[...], loop)

    pltpu.emit_pipeline(
        body,
        grid=(N_ELEMS // CHUNK,),
        in_specs=[
            pl.BlockSpec((CHUNK // 512, 512), lambda i: (i, 0)),
            pl.BlockSpec((CHUNK // 512, 512), lambda i: (i, 0)),
        ],
        core_axis_name=("c", "s"),
        dimension_semantics=(pltpu.PARALLEL,),
    )(x_hbm, t_hbm)

    pltpu.sync_copy(cnt_v, cnt_out.at[wid])
    pltpu.sync_copy(sm_v, sm_out.at[wid])
    pltpu.sync_copy(mx_v, mx_out.at[wid])


def _shift_down(x, k, axis):
    pads = [(0, 0), (0, 0)]
    pads[axis] = (k, 0)
    padded = jnp.pad(x, pads)
    if axis == 0:
        return padded[: x.shape[0], :]
    return padded[:, : x.shape[1]]


def _prefix2d(x):
    for k in (1, 2, 4, 8, 16, 32, 64):
        x = x + _shift_down(x, k, 1)
    rt = x[:, 127:128]
    rti = rt
    k = 1
    while k < x.shape[0]:
        rti = rti + _shift_down(rti, k, 0)
        k *= 2
    return x + (rti - rt)


def _tc_finish_kernel(cnt_ref, sm_ref, mx_ref, out_ref):
    cnt = jnp.sum(cnt_ref[...], axis=0)
    sm = jnp.sum(sm_ref[...], axis=0)
    cn = cnt[:64, :]
    cp = cnt[64:, :]
    sn = sm[:64, :]
    sp = sm[64:, :]
    pref_cn = _prefix2d(cn)
    pref_cp = _prefix2d(cp)
    p_tot = jnp.sum(cp)
    n_tot = jnp.sum(cn)
    ngt = n_tot - pref_cn
    pge = p_tot - pref_cp + cp
    a = p_tot + ngt
    contrib = sp / a + sn * (p_tot - pge) / (a * (a + cn))
    loss = jnp.sum(contrib)
    maxr = jnp.max(mx_ref[...])
    out = jnp.where(p_tot > 0.0, loss, maxr)
    out_ref[...] = out * jnp.ones((1, 1), jnp.float32)


_SC_PARAMS = pltpu.CompilerParams()
if "needs_layout_passes" in pltpu.CompilerParams.__dataclass_fields__:
    _SC_PARAMS = dataclasses.replace(_SC_PARAMS, needs_layout_passes=False)
_SC_PARAMS = dataclasses.replace(_SC_PARAMS, use_tc_tiling_on_sc=True)


@functools.partial(
    pl.kernel,
    compiler_params=_SC_PARAMS,
    out_type=[
        jax.ShapeDtypeStruct((NW, HSIZE), jnp.float32),
        jax.ShapeDtypeStruct((NW, HSIZE), jnp.float32),
        jax.ShapeDtypeStruct((NW, LANES), jnp.float32),
    ],
    mesh=plsc.VectorSubcoreMesh(core_axis_name="c", subcore_axis_name="s"),
    scratch_types=[
        pltpu.VMEM((HSIZE,), jnp.float32),
        pltpu.VMEM((HSIZE,), jnp.float32),
        pltpu.VMEM((LANES,), jnp.float32),
    ],
)
def _sc_hist(x_hbm, t_hbm, cnt_out, sm_out, mx_out, cnt_v, sm_v, mx_v):
    _sc_hist_kernel(x_hbm, t_hbm, cnt_out, sm_out, mx_out, cnt_v, sm_v, mx_v)


_tc_finish = pl.pallas_call(
    _tc_finish_kernel,
    out_shape=jax.ShapeDtypeStruct((1, 1), jnp.float32),
)


def kernel(inputs, targets):
    x = inputs.reshape(-1, 512)
    t = targets.reshape(-1, 512)
    cnt, sm, mx = _sc_hist(x, t)
    loss = _tc_finish(
        cnt.reshape(NW, 128, 128), sm.reshape(NW, 128, 128), mx
    )
    return loss[0, 0]
